# Initial kernel scaffold; baseline (speedup 1.0000x reference)
import jax
import jax.numpy as jnp
from jax.experimental import pallas as pl


def kernel(x, W):
    raise NotImplementedError("write your pallas kernel here")


# speedup vs baseline: 4.0845x; 4.0845x over previous
import functools

import jax
import jax.numpy as jnp
from jax import lax
from jax.experimental import pallas as pl
from jax.experimental.pallas import tpu as pltpu
from jax.experimental.pallas import tpu_sc as plsc

NUM_CORES = 2
NUM_SUBCORES = 16
NW = NUM_CORES * NUM_SUBCORES
CHUNK = 128


@functools.partial(jax.jit, static_argnames=("n_rows", "emb"))
def _sc_gather(W, xf, *, n_rows, emb):
    rows_per_w = n_rows // NW
    chunks_per_w = rows_per_w // CHUNK
    mesh = plsc.VectorSubcoreMesh(core_axis_name="c", subcore_axis_name="s")

    @functools.partial(
        pl.kernel,
        mesh=mesh,
        out_type=jax.ShapeDtypeStruct((n_rows, emb), jnp.float32),
        scratch_types=[
            pltpu.VMEM((rows_per_w,), jnp.int32),
            pltpu.VMEM((CHUNK, emb), jnp.float32),
            pltpu.SemaphoreType.DMA,
        ],
        compiler_params=pltpu.CompilerParams(use_tc_tiling_on_sc=False),
    )
    def k(W_hbm, xf_hbm, out_hbm, idx_v, rows_v, sem):
        wid = lax.axis_index("s") * NUM_CORES + lax.axis_index("c")
        base = wid * rows_per_w
        pltpu.sync_copy(xf_hbm.at[pl.ds(base, rows_per_w)], idx_v)

        def body(j, carry):
            idx = idx_v.at[pl.ds(j * CHUNK, CHUNK)]
            pltpu.async_copy(W_hbm.at[idx], rows_v, sem).wait()
            pltpu.sync_copy(rows_v, out_hbm.at[pl.ds(base + j * CHUNK, CHUNK)])
            return carry

        lax.fori_loop(0, chunks_per_w, body, 0)

    return k(W, xf)


def kernel(x, W):
    b, s = x.shape
    n_rows = b * s
    xf = x.reshape(n_rows).astype(jnp.int32)
    out = _sc_gather(W, xf, n_rows=n_rows, emb=W.shape[1])
    return out.reshape(b, s, W.shape[1])


# speedup vs baseline: 4.6557x; 1.1399x over previous
import functools

import jax
import jax.numpy as jnp
from jax import lax
from jax.experimental import pallas as pl
from jax.experimental.pallas import tpu as pltpu
from jax.experimental.pallas import tpu_sc as plsc

NUM_CORES = 2
NUM_SUBCORES = 16
NW = NUM_CORES * NUM_SUBCORES
CHUNK = 128


@functools.partial(jax.jit, static_argnames=("n_rows", "emb"))
def _sc_gather(W, xf, *, n_rows, emb):
    rows_per_w = n_rows // NW
    chunks_per_w = rows_per_w // CHUNK
    nbuf = 5
    outer = chunks_per_w // nbuf
    mesh = plsc.VectorSubcoreMesh(core_axis_name="c", subcore_axis_name="s")

    @functools.partial(
        pl.kernel,
        mesh=mesh,
        out_type=jax.ShapeDtypeStruct((n_rows, emb), jnp.float32),
        scratch_types=[
            pltpu.VMEM((rows_per_w,), jnp.int32),
            pltpu.VMEM((nbuf, CHUNK, emb), jnp.float32),
            pltpu.SemaphoreType.DMA((nbuf,)),
            pltpu.SemaphoreType.DMA((nbuf,)),
        ],
        compiler_params=pltpu.CompilerParams(use_tc_tiling_on_sc=False),
    )
    def k(W_hbm, xf_hbm, out_hbm, idx_v, rows_v, gsem, wsem):
        wid = lax.axis_index("s") * NUM_CORES + lax.axis_index("c")
        base = wid * rows_per_w
        pltpu.sync_copy(xf_hbm.at[pl.ds(base, rows_per_w)], idx_v)

        def gather_desc(j, b):
            idx = idx_v.at[pl.ds(j * CHUNK, CHUNK)]
            return pltpu.make_async_copy(W_hbm.at[idx], rows_v.at[b], gsem.at[b])

        def wb_desc(j, b):
            dst = out_hbm.at[pl.ds(base + j * CHUNK, CHUNK)]
            return pltpu.make_async_copy(rows_v.at[b], dst, wsem.at[b])

        for b in range(nbuf):
            gather_desc(b, b).start()

        def body(t, carry):
            j0 = t * nbuf
            for b in range(nbuf):
                gather_desc(j0 + b, b).wait()
                wb_desc(j0 + b, b).start()

            @pl.when(t < outer - 1)
            def _rearm():
                for b in range(nbuf):
                    wb_desc(j0 + b, b).wait()
                    gather_desc(j0 + nbuf + b, b).start()

            return carry

        lax.fori_loop(0, outer, body, 0)
        jlast = (outer - 1) * nbuf
        for b in range(nbuf):
            wb_desc(jlast + b, b).wait()

    return k(W, xf)


def kernel(x, W):
    b, s = x.shape
    n_rows = b * s
    xf = x.reshape(n_rows).astype(jnp.int32)
    out = _sc_gather(W, xf, n_rows=n_rows, emb=W.shape[1])
    return out.reshape(b, s, W.shape[1])
